# Initial kernel scaffold; baseline (speedup 1.0000x reference)
import jax
import jax.numpy as jnp
from jax import lax
from jax.experimental import pallas as pl
from jax.experimental.pallas import tpu as pltpu
from jax.experimental.pallas import tpu_sc as plsc

B = 4096
H = 50
D = 64
NC = 2
NS = 16
NW = NC * NS
RPW = B // NW
C = 2
G = RPW // C
IPC = C * H
IPCP = 104
L = 16
NCOL = D // L


def _body(ids_g, ids_t, emb, out, idx_v, idt_v, e0_v, nzf_v, rcp_v,
          buf_v, out_v, sem0, sem1):
    wid = lax.axis_index("s") * NC + lax.axis_index("c")

    pltpu.sync_copy(ids_g.at[wid], idx_v)
    pltpu.async_copy(emb.at[idx_v.at[0]], buf_v.at[0], sem0)
    pltpu.async_copy(emb.at[idx_v.at[1]], buf_v.at[1], sem1)
    pltpu.sync_copy(ids_t.at[wid], idt_v)
    pltpu.sync_copy(emb.at[0], e0_v)

    for v in range(RPW // L):
        acc = jnp.zeros((L,), jnp.int32)
        for h in range(H):
            x = idt_v[h, pl.ds(v * L, L)]
            acc = acc + jnp.where(x == 0, 1, 0).astype(jnp.int32)
        nzf = acc.astype(jnp.float32)
        nzf_v[pl.ds(v * L, L)] = nzf
        rcp_v[pl.ds(v * L, L)] = 1.0 / jnp.maximum(float(H) - nzf, 1.0)

    e0 = [e0_v[pl.ds(c * L, L)] for c in range(NCOL)]
    sems = (sem0, sem1)

    def step(s, carry):
        for bslot in range(2):
            g = 2 * s + bslot
            pltpu.make_async_copy(
                emb.at[pl.ds(0, IPCP)], buf_v.at[bslot], sems[bslot]).wait()
            bufs = buf_v.at[bslot]
            for r in range(C):
                row = C * g + r
                nzf = nzf_v[row]
                rcp = rcp_v[row]
                accs = [bufs[r * H, pl.ds(c * L, L)] for c in range(NCOL)]
                for j in range(1, H):
                    for c in range(NCOL):
                        accs[c] = accs[c] + bufs[r * H + j, pl.ds(c * L, L)]
                for c in range(NCOL):
                    out_v[row, pl.ds(c * L, L)] = (accs[c] - nzf * e0[c]) * rcp

            @pl.when(s < G // 2 - 1)
            def _():
                pltpu.async_copy(
                    emb.at[idx_v.at[g + 2]], buf_v.at[bslot], sems[bslot])
        return carry

    lax.fori_loop(0, G // 2, step, None)
    pltpu.sync_copy(out_v, out.at[pl.ds(wid * RPW, RPW)])


@jax.jit
def _run(ids32, embeddings):
    ids_g = jnp.pad(ids32.reshape(NW, G, IPC), ((0, 0), (0, 0), (0, IPCP - IPC)))
    ids_t = ids32.reshape(NW, RPW, H).transpose(0, 2, 1)
    kfn = pl.kernel(
        _body,
        out_type=jax.ShapeDtypeStruct((B, D), jnp.float32),
        mesh=plsc.VectorSubcoreMesh(core_axis_name="c", subcore_axis_name="s"),
        scratch_types=[
            pltpu.VMEM((G, IPCP), jnp.int32),
            pltpu.VMEM((H, RPW), jnp.int32),
            pltpu.VMEM((D,), jnp.float32),
            pltpu.VMEM((RPW,), jnp.float32),
            pltpu.VMEM((RPW,), jnp.float32),
            pltpu.VMEM((2, IPCP, D), jnp.float32),
            pltpu.VMEM((RPW, D), jnp.float32),
            pltpu.SemaphoreType.DMA,
            pltpu.SemaphoreType.DMA,
        ],
    )
    return kfn(ids_g, ids_t, embeddings)


def kernel(inputs, embeddings):
    return _run(inputs.astype(jnp.int32), embeddings)


# speedup vs baseline: 3.9968x; 3.9968x over previous
import jax
import jax.numpy as jnp
from jax import lax
from jax.experimental import pallas as pl
from jax.experimental.pallas import tpu as pltpu
from jax.experimental.pallas import tpu_sc as plsc

B = 4096
H = 50
D = 64
NC = 2
NS = 16
NW = NC * NS
RPW = B // NW
C = 2
G = RPW // C
IPC = C * H
IPCP = 104
L = 16
NCOL = D // L


def _body(ids_g, ids_t, emb, out, idx_v, idt_v, e0_v, nzf_v, rcp_v,
          buf_v, out_v, sem0, sem1):
    wid = lax.axis_index("s") * NC + lax.axis_index("c")

    pltpu.sync_copy(ids_g.at[wid], idx_v)
    pltpu.async_copy(emb.at[idx_v.at[0]], buf_v.at[0], sem0)
    pltpu.async_copy(emb.at[idx_v.at[1]], buf_v.at[1], sem1)
    pltpu.sync_copy(ids_t.at[wid], idt_v)
    pltpu.sync_copy(emb.at[0], e0_v)

    for v in range(RPW // L):
        acc = jnp.zeros((L,), jnp.int32)
        for h in range(H):
            x = idt_v[h, pl.ds(v * L, L)]
            acc = acc + jnp.where(x == 0, 1, 0).astype(jnp.int32)
        nzf = acc.astype(jnp.float32)
        nzf_v[pl.ds(v * L, L)] = nzf
        rcp_v[pl.ds(v * L, L)] = 1.0 / jnp.maximum(float(H) - nzf, 1.0)

    e0 = [e0_v[pl.ds(c * L, L)] for c in range(NCOL)]
    sems = (sem0, sem1)

    def step(s, carry):
        for bslot in range(2):
            g = 2 * s + bslot
            pltpu.make_async_copy(
                emb.at[pl.ds(0, IPCP)], buf_v.at[bslot], sems[bslot]).wait()
            bufs = buf_v.at[bslot]
            for r in range(C):
                row = C * g + r
                rowv = jnp.full((L,), row, jnp.int32)
                nzf = plsc.load_gather(nzf_v, [rowv])
                rcp = plsc.load_gather(rcp_v, [rowv])
                accs = [bufs[r * H, pl.ds(c * L, L)] for c in range(NCOL)]
                for j in range(1, H):
                    for c in range(NCOL):
                        accs[c] = accs[c] + bufs[r * H + j, pl.ds(c * L, L)]
                for c in range(NCOL):
                    out_v[row, pl.ds(c * L, L)] = (accs[c] - nzf * e0[c]) * rcp

            @pl.when(s < G // 2 - 1)
            def _():
                pltpu.async_copy(
                    emb.at[idx_v.at[g + 2]], buf_v.at[bslot], sems[bslot])
        return carry

    lax.fori_loop(0, G // 2, step, None)
    pltpu.sync_copy(out_v, out.at[pl.ds(wid * RPW, RPW)])


@jax.jit
def _run(ids32, embeddings):
    ids_g = jnp.pad(ids32.reshape(NW, G, IPC), ((0, 0), (0, 0), (0, IPCP - IPC)))
    ids_t = ids32.reshape(NW, RPW, H).transpose(0, 2, 1)
    kfn = pl.kernel(
        _body,
        out_type=jax.ShapeDtypeStruct((B, D), jnp.float32),
        mesh=plsc.VectorSubcoreMesh(core_axis_name="c", subcore_axis_name="s",
                                    num_cores=NC, num_subcores=NS),
        scratch_types=[
            pltpu.VMEM((G, IPCP), jnp.int32),
            pltpu.VMEM((H, RPW), jnp.int32),
            pltpu.VMEM((D,), jnp.float32),
            pltpu.VMEM((RPW,), jnp.float32),
            pltpu.VMEM((RPW,), jnp.float32),
            pltpu.VMEM((2, IPCP, D), jnp.float32),
            pltpu.VMEM((RPW, D), jnp.float32),
            pltpu.SemaphoreType.DMA,
            pltpu.SemaphoreType.DMA,
        ],
        compiler_params=pltpu.CompilerParams(needs_layout_passes=False,
                                             use_tc_tiling_on_sc=False),
    )
    return kfn(ids_g, ids_t, embeddings)


def kernel(inputs, embeddings):
    return _run(inputs.astype(jnp.int32), embeddings)


# speedup vs baseline: 4.3898x; 1.0983x over previous
import jax
import jax.numpy as jnp
import numpy as np
from jax import lax
from jax.experimental import pallas as pl
from jax.experimental.pallas import tpu as pltpu
from jax.experimental.pallas import tpu_sc as plsc

B = 4096
H = 50
D = 64
DH = D // 2
V = 100000
VP = 100096
NC = 2
NS = 16
RPT = B // NS
C = 2
G = RPT // C
IPC = C * H
IPCP = 104
L = 16
NBUF = 4
VPT = VP // NS


def _body(ids_g, emb_a, emb_b, out, tab_v, idx_v, buf_v, out_v, zrow_v,
          *sems):
    c = lax.axis_index("c")
    s = lax.axis_index("s")

    @pl.when(c == 0)
    def _():
        pltpu.sync_copy(emb_a.at[pl.ds(s * VPT, VPT)],
                        tab_v.at[pl.ds(s * VPT, VPT)])

    @pl.when(c == 1)
    def _():
        pltpu.sync_copy(emb_b.at[pl.ds(s * VPT, VPT)],
                        tab_v.at[pl.ds(s * VPT, VPT)])

    pltpu.sync_copy(ids_g.at[s], idx_v)
    plsc.subcore_barrier()

    for b in range(NBUF):
        pltpu.async_copy(tab_v.at[idx_v.at[b]], buf_v.at[b], sems[b])

    pltpu.sync_copy(tab_v.at[pl.ds(0, 8)], zrow_v)
    e0a, e0b = plsc.unpack(zrow_v[0, pl.ds(0, 2 * L)],
                           format=plsc.PackFormat.INTERLEAVED)

    iota = lax.iota(jnp.int32, L)
    m_r0 = iota < 2
    m_r1 = iota >= 2
    m_tl = (iota >= 8) & (iota < 12)
    one = jnp.ones((L,), jnp.int32)
    zero = jnp.zeros((L,), jnp.int32)

    def step(sg, carry):
        for bslot in range(NBUF):
            g = NBUF * sg + bslot
            z = [jnp.where(idx_v[g, pl.ds(off, L)] == 0, one, zero)
                 for off in (0, 16, 32, 48, 64, 80, 88)]
            nz0 = z[0] + z[1] + z[2] + jnp.where(m_r0, z[3], zero)
            nz1 = (jnp.where(m_r1, z[3], zero) + z[4] + z[5]
                   + jnp.where(m_tl, z[6], zero))
            nzf = (jnp.sum(nz0).astype(jnp.float32),
                   jnp.sum(nz1).astype(jnp.float32))

            pltpu.make_async_copy(
                emb_a.at[pl.ds(0, IPCP)], buf_v.at[bslot], sems[bslot]).wait()
            bufs = buf_v.at[bslot]
            for r in range(C):
                row = C * g + r
                x0 = bufs[r * H, pl.ds(0, 2 * L)]
                acc_a, acc_b = plsc.unpack(
                    x0, format=plsc.PackFormat.INTERLEAVED)
                for j in range(1, H):
                    x = bufs[r * H + j, pl.ds(0, 2 * L)]
                    xa, xb = plsc.unpack(x, format=plsc.PackFormat.INTERLEAVED)
                    acc_a = acc_a + xa
                    acc_b = acc_b + xb
                nzv = jnp.full((L,), nzf[r], jnp.float32)
                rcp = 1.0 / jnp.maximum(float(H) - nzv, 1.0)
                out_v[row, pl.ds(0, L)] = (acc_a - nzv * e0a) * rcp
                out_v[row, pl.ds(L, L)] = (acc_b - nzv * e0b) * rcp

            @pl.when(sg < G // NBUF - 1)
            def _():
                pltpu.async_copy(
                    tab_v.at[idx_v.at[g + NBUF]], buf_v.at[bslot], sems[bslot])
        return carry

    lax.fori_loop(0, G // NBUF, step, None)
    pltpu.sync_copy(out_v, out.at[pl.ds(s * RPT, RPT), pl.ds(c * DH, DH)])


_PERM = np.stack([np.arange(16), np.arange(16, 32)], axis=1).reshape(-1)


@jax.jit
def _run(ids32, embeddings):
    ids_g = jnp.pad(ids32.reshape(NS, G, IPC),
                    ((0, 0), (0, 0), (0, IPCP - IPC)))
    pad = jnp.zeros((VP - V, DH), jnp.bfloat16)
    emb_a = jnp.concatenate(
        [embeddings[:, :DH][:, _PERM].astype(jnp.bfloat16), pad])
    emb_b = jnp.concatenate(
        [embeddings[:, DH:][:, _PERM].astype(jnp.bfloat16), pad])
    kfn = pl.kernel(
        _body,
        out_type=jax.ShapeDtypeStruct((B, D), jnp.float32),
        mesh=plsc.VectorSubcoreMesh(core_axis_name="c", subcore_axis_name="s",
                                    num_cores=NC, num_subcores=NS),
        scratch_types=[
            pltpu.VMEM_SHARED((VP, DH), jnp.bfloat16),
            pltpu.VMEM((G, IPCP), jnp.int32),
            pltpu.VMEM((NBUF, IPCP, DH), jnp.bfloat16),
            pltpu.VMEM((RPT, DH), jnp.float32),
            pltpu.VMEM((8, DH), jnp.bfloat16),
        ] + [pltpu.SemaphoreType.DMA] * NBUF,
        compiler_params=pltpu.CompilerParams(needs_layout_passes=False,
                                             use_tc_tiling_on_sc=False),
    )
    return kfn(ids_g, emb_a, emb_b)


def kernel(inputs, embeddings):
    return _run(inputs.astype(jnp.int32), embeddings)
